# baseline (speedup 1.0000x reference)
import functools

import jax
import jax.numpy as jnp
from jax import lax
from jax.experimental import pallas as pl
from jax.experimental.pallas import tpu as pltpu
from jax.experimental.pallas import tpu_sc as plsc

_EMBED = 32
_BATCH = 4096
_TILE = 512
_NB = _BATCH // _TILE
_NPAIRS = _BATCH * (_BATCH - 1) / 2.0
_GAMMA = 1.0
_REG_LAMBDA = 0.001


def _gather_sc(user_table, item_table, user_idx, pos_idx):
    info = plsc.get_sparse_core_info()
    nc, ns = info.num_cores, info.num_subcores
    nw = nc * ns
    bpw = _BATCH // nw
    mesh = plsc.VectorSubcoreMesh(core_axis_name="c", subcore_axis_name="s")

    @functools.partial(
        pl.kernel,
        mesh=mesh,
        compiler_params=pltpu.CompilerParams(use_tc_tiling_on_sc=False),
        out_type=(
            jax.ShapeDtypeStruct((_BATCH, _EMBED), jnp.float32),
            jax.ShapeDtypeStruct((_BATCH, _EMBED), jnp.float32),
        ),
        scratch_types=[
            pltpu.VMEM((bpw,), jnp.int32),
            pltpu.VMEM((bpw, _EMBED), jnp.float32),
            pltpu.SemaphoreType.DMA,
        ],
    )
    def gk(ut_hbm, it_hbm, ui_hbm, pi_hbm, uo_hbm, po_hbm, idx_v, rows_v, sem):
        wid = lax.axis_index("s") * nc + lax.axis_index("c")
        base = wid * bpw
        pltpu.sync_copy(ui_hbm.at[pl.ds(base, bpw)], idx_v)
        pltpu.async_copy(ut_hbm.at[idx_v], rows_v, sem).wait()
        pltpu.sync_copy(rows_v, uo_hbm.at[pl.ds(base, bpw)])
        pltpu.sync_copy(pi_hbm.at[pl.ds(base, bpw)], idx_v)
        pltpu.async_copy(it_hbm.at[idx_v], rows_v, sem).wait()
        pltpu.sync_copy(rows_v, po_hbm.at[pl.ds(base, bpw)])

    return gk(user_table, item_table, user_idx, pos_idx)


def _normalize(x):
    n = jnp.sqrt(jnp.sum(x * x, axis=1, keepdims=True))
    return x / jnp.maximum(n, 1e-12)


def _loss_body(u_ref, p_ref, out_ref, acc_ref):
    bi = pl.program_id(0)
    bj = pl.program_id(1)

    @pl.when((bi == 0) & (bj == 0))
    def _init():
        acc_ref[0] = 0.0
        acc_ref[1] = 0.0
        acc_ref[2] = 0.0
        acc_ref[3] = 0.0

    @pl.when(bj >= bi)
    def _work():
        ui = u_ref[pl.ds(bi * _TILE, _TILE), :]
        pi = p_ref[pl.ds(bi * _TILE, _TILE), :]
        uj = u_ref[pl.ds(bj * _TILE, _TILE), :]
        pj = p_ref[pl.ds(bj * _TILE, _TILE), :]
        nui, npi = _normalize(ui), _normalize(pi)
        nuj, npj = _normalize(uj), _normalize(pj)

        def pair_sum(ai, aj):
            sqi = jnp.sum(ai * ai, axis=1, keepdims=True)
            sqj = jnp.sum(aj * aj, axis=1, keepdims=True)
            ones = jnp.ones((_TILE, 1), jnp.float32)
            a = jnp.concatenate([ai, sqi, ones], axis=1)
            b = jnp.concatenate([-2.0 * aj, ones, sqj], axis=1)
            d2 = lax.dot_general(a, b, (((1,), (1,)), ((), ())),
                                 preferred_element_type=jnp.float32)
            e = jnp.exp(-2.0 * jnp.maximum(d2, 0.0))
            s = jnp.sum(e)
            r = lax.broadcasted_iota(jnp.int32, (_TILE, _TILE), 0)
            c = lax.broadcasted_iota(jnp.int32, (_TILE, _TILE), 1)
            tr = jnp.sum(jnp.where(r == c, e, 0.0))
            return jnp.where(bi == bj, 0.5 * (s - tr), s)

        acc_ref[2] += pair_sum(nui, nuj)
        acc_ref[3] += pair_sum(npi, npj)

        @pl.when(bi == bj)
        def _diag():
            acc_ref[0] += jnp.sum((nui - npi) ** 2)
            acc_ref[1] += jnp.sum(ui * ui) + jnp.sum(pi * pi)

    @pl.when((bi == _NB - 1) & (bj == _NB - 1))
    def _final():
        col = lax.broadcasted_iota(jnp.int32, (1, 128), 1)
        align_v = jnp.full((1, 128), acc_ref[0] / _BATCH, jnp.float32)
        reg_v = jnp.full((1, 128), _REG_LAMBDA * 0.5 * acc_ref[1] / _BATCH,
                         jnp.float32)
        lu = jnp.log(jnp.full((1, 128), acc_ref[2] / _NPAIRS, jnp.float32))
        lp = jnp.log(jnp.full((1, 128), acc_ref[3] / _NPAIRS, jnp.float32))
        uni_v = _GAMMA * 0.5 * (lu + lp)
        out_ref[...] = jnp.where(col == 0, align_v,
                                 jnp.where(col == 1, uni_v, reg_v))


def _losses_tc(u_rows, p_rows):
    out = pl.pallas_call(
        _loss_body,
        grid=(_NB, _NB),
        in_specs=[
            pl.BlockSpec((_BATCH, _EMBED), lambda i, j: (0, 0)),
            pl.BlockSpec((_BATCH, _EMBED), lambda i, j: (0, 0)),
        ],
        out_specs=pl.BlockSpec((1, 128), lambda i, j: (0, 0)),
        out_shape=jax.ShapeDtypeStruct((1, 128), jnp.float32),
        scratch_shapes=[pltpu.SMEM((4,), jnp.float32)],
    )(u_rows, p_rows)
    return out[0, 0], out[0, 1], out[0, 2]


def kernel(user, positive, negative, user_table, item_table):
    del negative
    u_rows, p_rows = _gather_sc(
        user_table, item_table,
        user.astype(jnp.int32), positive.astype(jnp.int32),
    )
    return _losses_tc(u_rows, p_rows)


# speedup vs baseline: 1.0058x; 1.0058x over previous
import functools

import jax
import jax.numpy as jnp
from jax import lax
from jax.experimental import pallas as pl
from jax.experimental.pallas import tpu as pltpu
from jax.experimental.pallas import tpu_sc as plsc

_EMBED = 32
_BATCH = 4096
_TILE = 512
_NB = _BATCH // _TILE
_NPAIRS = _BATCH * (_BATCH - 1) / 2.0
_GAMMA = 1.0
_REG_LAMBDA = 0.001


_GRP = 4


def _gather_sc(ut_lin, it_lin, user_idx, pos_idx):
    info = plsc.get_sparse_core_info()
    nc, ns = info.num_cores, info.num_subcores
    nw = nc * ns
    bpw = _BATCH // nw
    chunk = 64
    mesh = plsc.VectorSubcoreMesh(core_axis_name="c", subcore_axis_name="s")

    @functools.partial(
        pl.kernel,
        mesh=mesh,
        out_type=(
            jax.ShapeDtypeStruct((_BATCH, 128), jnp.float32),
            jax.ShapeDtypeStruct((_BATCH, 128), jnp.float32),
        ),
        scratch_types=[
            pltpu.VMEM((bpw,), jnp.int32),
            pltpu.VMEM((bpw,), jnp.int32),
            pltpu.VMEM((chunk, 128), jnp.float32),
            pltpu.SemaphoreType.DMA,
        ],
    )
    def gk(ut_hbm, it_hbm, ui_hbm, pi_hbm, uo_hbm, po_hbm,
           idx_v, div_v, rows_v, sem):
        wid = lax.axis_index("s") * nc + lax.axis_index("c")
        base = wid * bpw
        for idx_hbm, tab_hbm, out_hbm in (
            (ui_hbm, ut_hbm, uo_hbm),
            (pi_hbm, it_hbm, po_hbm),
        ):
            pltpu.sync_copy(idx_hbm.at[pl.ds(base, bpw)], idx_v)
            for t in range(bpw // 16):
                v = idx_v[pl.ds(t * 16, 16)]
                div_v[pl.ds(t * 16, 16)] = lax.shift_right_logical(v, 2)
            for c in range(bpw // chunk):
                pltpu.async_copy(
                    tab_hbm.at[div_v.at[pl.ds(c * chunk, chunk)]],
                    rows_v, sem).wait()
                pltpu.sync_copy(rows_v,
                                out_hbm.at[pl.ds(base + c * chunk, chunk)])

    return gk(ut_lin, it_lin, user_idx, pos_idx)


def _select_body(su_ref, sp_ref, ui_ref, pi_ref,
                 xnu_ref, xnp_ref, sums_ref, acc_ref):
    j = pl.program_id(0)

    @pl.when(j == 0)
    def _init():
        acc_ref[0] = 0.0
        acc_ref[1] = 0.0

    def select(s_ref, i_ref):
        rem = i_ref[...] & (_GRP - 1)
        x = s_ref[...]
        emb = jnp.zeros((_TILE, _EMBED), jnp.float32)
        for r in range(_GRP):
            emb = jnp.where(rem == r, x[:, r * _EMBED:(r + 1) * _EMBED], emb)
        return emb

    emb_u = select(su_ref, ui_ref)
    emb_p = select(sp_ref, pi_ref)

    def normalize(x):
        n = jnp.sqrt(jnp.sum(x * x, axis=1, keepdims=True))
        return x / jnp.maximum(n, 1e-12)

    nu = normalize(emb_u)
    np_ = normalize(emb_p)
    xnu_ref[...] = nu
    xnp_ref[...] = np_
    acc_ref[0] += jnp.sum((nu - np_) ** 2)
    acc_ref[1] += jnp.sum(emb_u * emb_u) + jnp.sum(emb_p * emb_p)

    @pl.when(j == _NB - 1)
    def _final():
        sums_ref[0] = acc_ref[0]
        sums_ref[1] = acc_ref[1]


def _pair_body(xnu_ref, xnp_ref, sums_ref, out_ref, acc_ref):
    bi = pl.program_id(0)
    bj = pl.program_id(1)

    @pl.when((bi == 0) & (bj == 0))
    def _init():
        acc_ref[0] = 0.0
        acc_ref[1] = 0.0

    @pl.when(bj >= bi)
    def _work():
        for slot, x_ref in ((0, xnu_ref), (1, xnp_ref)):
            xi = x_ref[pl.ds(bi * _TILE, _TILE), :]
            xj = x_ref[pl.ds(bj * _TILE, _TILE), :]
            g = lax.dot_general(xi, xj, (((1,), (1,)), ((), ())),
                                preferred_element_type=jnp.float32)
            e = jnp.exp(-2.0 * jnp.maximum(2.0 - 2.0 * g, 0.0))
            s = jnp.sum(e)
            acc_ref[slot] += jnp.where(bi == bj, 0.5 * (s - _TILE), s)

    @pl.when((bi == _NB - 1) & (bj == _NB - 1))
    def _final():
        col = lax.broadcasted_iota(jnp.int32, (1, 128), 1)
        align_v = jnp.full((1, 128), sums_ref[0] / _BATCH, jnp.float32)
        reg_v = jnp.full((1, 128),
                         _REG_LAMBDA * 0.5 * sums_ref[1] / _BATCH, jnp.float32)
        lu = jnp.log(jnp.full((1, 128), acc_ref[0] / _NPAIRS, jnp.float32))
        lp = jnp.log(jnp.full((1, 128), acc_ref[1] / _NPAIRS, jnp.float32))
        uni_v = _GAMMA * 0.5 * (lu + lp)
        out_ref[...] = jnp.where(col == 0, align_v,
                                 jnp.where(col == 1, uni_v, reg_v))


def _losses_tc(slabs_u, slabs_p, uidx, pidx):
    xnu, xnp_, sums = pl.pallas_call(
        _select_body,
        grid=(_NB,),
        in_specs=[
            pl.BlockSpec((_TILE, 128), lambda j: (j, 0)),
            pl.BlockSpec((_TILE, 128), lambda j: (j, 0)),
            pl.BlockSpec((_TILE, 1), lambda j: (j, 0)),
            pl.BlockSpec((_TILE, 1), lambda j: (j, 0)),
        ],
        out_specs=(
            pl.BlockSpec((_TILE, _EMBED), lambda j: (j, 0)),
            pl.BlockSpec((_TILE, _EMBED), lambda j: (j, 0)),
            pl.BlockSpec(memory_space=pltpu.SMEM),
        ),
        out_shape=(
            jax.ShapeDtypeStruct((_BATCH, _EMBED), jnp.float32),
            jax.ShapeDtypeStruct((_BATCH, _EMBED), jnp.float32),
            jax.ShapeDtypeStruct((2,), jnp.float32),
        ),
        scratch_shapes=[pltpu.SMEM((2,), jnp.float32)],
    )(slabs_u, slabs_p, uidx, pidx)

    out = pl.pallas_call(
        _pair_body,
        grid=(_NB, _NB),
        in_specs=[
            pl.BlockSpec((_BATCH, _EMBED), lambda i, j: (0, 0)),
            pl.BlockSpec((_BATCH, _EMBED), lambda i, j: (0, 0)),
            pl.BlockSpec(memory_space=pltpu.SMEM),
        ],
        out_specs=pl.BlockSpec((1, 128), lambda i, j: (0, 0)),
        out_shape=jax.ShapeDtypeStruct((1, 128), jnp.float32),
        scratch_shapes=[pltpu.SMEM((2,), jnp.float32)],
    )(xnu, xnp_, sums)
    return out[0, 0], out[0, 1], out[0, 2]


def kernel(user, positive, negative, user_table, item_table):
    del negative
    ui = user.astype(jnp.int32)
    pi = positive.astype(jnp.int32)
    slabs_u, slabs_p = _gather_sc(
        user_table.reshape(-1, 128), item_table.reshape(-1, 128), ui, pi)
    return _losses_tc(slabs_u, slabs_p,
                      ui.reshape(_BATCH, 1), pi.reshape(_BATCH, 1))


# speedup vs baseline: 2.2154x; 2.2026x over previous
import functools

import jax
import jax.numpy as jnp
from jax import lax
from jax.experimental import pallas as pl
from jax.experimental.pallas import tpu as pltpu
from jax.experimental.pallas import tpu_sc as plsc

_EMBED = 32
_BATCH = 4096
_TILE = 512
_NB = _BATCH // _TILE
_NPAIRS = _BATCH * (_BATCH - 1) / 2.0
_GAMMA = 1.0
_REG_LAMBDA = 0.001


_GRP = 8


def _gather_sc(ut3, it3, user_idx, pos_idx):
    info = plsc.get_sparse_core_info()
    nc, ns = info.num_cores, info.num_subcores
    nw = nc * ns
    bpw = _BATCH // nw
    chunk = 64
    mesh = plsc.VectorSubcoreMesh(core_axis_name="c", subcore_axis_name="s")

    @functools.partial(
        pl.kernel,
        mesh=mesh,
        out_type=(
            jax.ShapeDtypeStruct((_BATCH, _GRP, _EMBED), jnp.float32),
            jax.ShapeDtypeStruct((_BATCH, _GRP, _EMBED), jnp.float32),
        ),
        scratch_types=[
            pltpu.VMEM((bpw,), jnp.int32),
            pltpu.VMEM((chunk, _GRP, _EMBED), jnp.float32),
            pltpu.SemaphoreType.DMA,
        ],
    )
    def gk(ut_hbm, it_hbm, ui_hbm, pi_hbm, uo_hbm, po_hbm,
           idx_v, rows_v, sem):
        wid = lax.axis_index("s") * nc + lax.axis_index("c")
        base = wid * bpw
        lane = lax.iota(jnp.int32, 16)
        for idx_hbm, tab_hbm, out_hbm in (
            (ui_hbm, ut_hbm, uo_hbm),
            (pi_hbm, it_hbm, po_hbm),
        ):
            pltpu.sync_copy(idx_hbm.at[pl.ds(base, bpw)], idx_v)
            for c in range(bpw // chunk):
                cps = []
                for k in range(chunk):
                    kk = c * chunk + k
                    v = idx_v[pl.ds((kk // 16) * 16, 16)]
                    i = v[kk % 16]
                    s = lax.shift_right_logical(i, 3)
                    cps.append(pltpu.async_copy(
                        tab_hbm.at[s], rows_v.at[k], sem))
                for cp in cps:
                    cp.wait()
                pltpu.sync_copy(rows_v,
                                out_hbm.at[pl.ds(base + c * chunk, chunk)])

    return gk(ut3, it3, user_idx, pos_idx)


def _select_body(su_ref, sp_ref, ui_ref, pi_ref,
                 xnu_ref, xnp_ref, sums_ref, acc_ref):
    j = pl.program_id(0)

    @pl.when(j == 0)
    def _init():
        acc_ref[0] = 0.0
        acc_ref[1] = 0.0

    def select(s_ref, i_ref):
        rem = i_ref[...] & (_GRP - 1)
        x = s_ref[...]
        emb = jnp.zeros((_TILE, _EMBED), jnp.float32)
        for r in range(_GRP):
            emb = jnp.where(rem == r, x[:, r, :], emb)
        return emb

    emb_u = select(su_ref, ui_ref)
    emb_p = select(sp_ref, pi_ref)

    def normalize(x):
        n = jnp.sqrt(jnp.sum(x * x, axis=1, keepdims=True))
        return x / jnp.maximum(n, 1e-12)

    nu = normalize(emb_u)
    np_ = normalize(emb_p)
    xnu_ref[...] = nu
    xnp_ref[...] = np_
    acc_ref[0] += jnp.sum((nu - np_) ** 2)
    acc_ref[1] += jnp.sum(emb_u * emb_u) + jnp.sum(emb_p * emb_p)

    @pl.when(j == _NB - 1)
    def _final():
        sums_ref[0] = acc_ref[0]
        sums_ref[1] = acc_ref[1]


def _pair_body(xnu_ref, xnp_ref, sums_ref, out_ref, acc_ref):
    bi = pl.program_id(0)
    bj = pl.program_id(1)

    @pl.when((bi == 0) & (bj == 0))
    def _init():
        acc_ref[0] = 0.0
        acc_ref[1] = 0.0

    @pl.when(bj >= bi)
    def _work():
        for slot, x_ref in ((0, xnu_ref), (1, xnp_ref)):
            xi = x_ref[pl.ds(bi * _TILE, _TILE), :]
            xj = x_ref[pl.ds(bj * _TILE, _TILE), :]
            g = lax.dot_general(xi, xj, (((1,), (1,)), ((), ())),
                                preferred_element_type=jnp.float32)
            e = jnp.exp(-2.0 * jnp.maximum(2.0 - 2.0 * g, 0.0))
            s = jnp.sum(e)
            acc_ref[slot] += jnp.where(bi == bj, 0.5 * (s - _TILE), s)

    @pl.when((bi == _NB - 1) & (bj == _NB - 1))
    def _final():
        col = lax.broadcasted_iota(jnp.int32, (1, 128), 1)
        align_v = jnp.full((1, 128), sums_ref[0] / _BATCH, jnp.float32)
        reg_v = jnp.full((1, 128),
                         _REG_LAMBDA * 0.5 * sums_ref[1] / _BATCH, jnp.float32)
        lu = jnp.log(jnp.full((1, 128), acc_ref[0] / _NPAIRS, jnp.float32))
        lp = jnp.log(jnp.full((1, 128), acc_ref[1] / _NPAIRS, jnp.float32))
        uni_v = _GAMMA * 0.5 * (lu + lp)
        out_ref[...] = jnp.where(col == 0, align_v,
                                 jnp.where(col == 1, uni_v, reg_v))


def _losses_tc(slabs_u, slabs_p, uidx, pidx):
    xnu, xnp_, sums = pl.pallas_call(
        _select_body,
        grid=(_NB,),
        in_specs=[
            pl.BlockSpec((_TILE, _GRP, _EMBED), lambda j: (j, 0, 0)),
            pl.BlockSpec((_TILE, _GRP, _EMBED), lambda j: (j, 0, 0)),
            pl.BlockSpec((_TILE, 1), lambda j: (j, 0)),
            pl.BlockSpec((_TILE, 1), lambda j: (j, 0)),
        ],
        out_specs=(
            pl.BlockSpec((_TILE, _EMBED), lambda j: (j, 0)),
            pl.BlockSpec((_TILE, _EMBED), lambda j: (j, 0)),
            pl.BlockSpec(memory_space=pltpu.SMEM),
        ),
        out_shape=(
            jax.ShapeDtypeStruct((_BATCH, _EMBED), jnp.float32),
            jax.ShapeDtypeStruct((_BATCH, _EMBED), jnp.float32),
            jax.ShapeDtypeStruct((2,), jnp.float32),
        ),
        scratch_shapes=[pltpu.SMEM((2,), jnp.float32)],
    )(slabs_u, slabs_p, uidx, pidx)

    out = pl.pallas_call(
        _pair_body,
        grid=(_NB, _NB),
        in_specs=[
            pl.BlockSpec((_BATCH, _EMBED), lambda i, j: (0, 0)),
            pl.BlockSpec((_BATCH, _EMBED), lambda i, j: (0, 0)),
            pl.BlockSpec(memory_space=pltpu.SMEM),
        ],
        out_specs=pl.BlockSpec((1, 128), lambda i, j: (0, 0)),
        out_shape=jax.ShapeDtypeStruct((1, 128), jnp.float32),
        scratch_shapes=[pltpu.SMEM((2,), jnp.float32)],
    )(xnu, xnp_, sums)
    return out[0, 0], out[0, 1], out[0, 2]


def kernel(user, positive, negative, user_table, item_table):
    del negative
    ui = user.astype(jnp.int32)
    pi = positive.astype(jnp.int32)
    slabs_u, slabs_p = _gather_sc(
        user_table.reshape(-1, _GRP, _EMBED),
        item_table.reshape(-1, _GRP, _EMBED), ui, pi)
    return _losses_tc(slabs_u, slabs_p,
                      ui.reshape(_BATCH, 1), pi.reshape(_BATCH, 1))
